# Initial kernel scaffold; baseline (speedup 1.0000x reference)
import jax
import jax.numpy as jnp
from jax.experimental import pallas as pl


def kernel(doc, table):
    raise NotImplementedError("write your pallas kernel here")


# speedup vs baseline: 3.9558x; 3.9558x over previous
import functools

import jax
import jax.numpy as jnp
from jax import lax
from jax.experimental import pallas as pl
from jax.experimental.pallas import tpu as pltpu
from jax.experimental.pallas import tpu_sc as plsc

DIM = 64
MAX_NORM = 1.0


def _renorm_body(tab_ref, out_ref):
    x = tab_ref[...]
    norm = jnp.sqrt(jnp.sum(x * x, axis=1, keepdims=True))
    scale = jnp.where(norm > MAX_NORM, MAX_NORM / (norm + 1e-7), 1.0)
    out_ref[...] = x * scale


def _renorm_table(table):
    vocab = table.shape[0]
    grid = 10
    rb = vocab // grid
    return pl.pallas_call(
        _renorm_body,
        grid=(grid,),
        in_specs=[pl.BlockSpec((rb, DIM), lambda i: (i, 0))],
        out_specs=pl.BlockSpec((rb, DIM), lambda i: (i, 0)),
        out_shape=jax.ShapeDtypeStruct(table.shape, table.dtype),
    )(table)


_CHUNK = 128
_NBUF = 6
_LAG = _NBUF // 2


@functools.cache
def _make_gather(n_rows, vocab):
    info = plsc.get_sparse_core_info()
    nc, ns = info.num_cores, info.num_subcores
    nw = nc * ns
    n_chunks = n_rows // _CHUNK
    per_w = n_chunks // nw
    assert per_w * nw == n_chunks and n_chunks * _CHUNK == n_rows
    mesh = plsc.VectorSubcoreMesh(core_axis_name="c", subcore_axis_name="s")

    @functools.partial(
        pl.kernel,
        mesh=mesh,
        compiler_params=pltpu.CompilerParams(use_tc_tiling_on_sc=False),
        out_type=jax.ShapeDtypeStruct((n_rows, DIM), jnp.float32),
        scratch_types=(
            [pltpu.VMEM((per_w, _CHUNK), jnp.int32)]
            + [pltpu.VMEM((_CHUNK, DIM), jnp.float32) for _ in range(_NBUF)]
            + [pltpu.SemaphoreType.DMA for _ in range(2 * _NBUF)]
        ),
    )
    def gather_k(tab_hbm, idx_hbm, out_hbm, idx_v, *rest):
        bufs = rest[:_NBUF]
        gsems = rest[_NBUF:2 * _NBUF]
        wsems = rest[2 * _NBUF:]
        wid = lax.axis_index("s") * nc + lax.axis_index("c")
        cbase = wid * per_w
        pltpu.sync_copy(idx_hbm.at[wid], idx_v)

        hg = [None] * _NBUF
        hw = [None] * _NBUF
        for j in range(per_w + _LAG):
            if j < per_w:
                b = j % _NBUF
                if j >= _NBUF:
                    hw[b].wait()
                hg[b] = pltpu.async_copy(
                    tab_hbm.at[idx_v.at[j]], bufs[b], gsems[b])
            i = j - _LAG
            if 0 <= i < per_w:
                bi = i % _NBUF
                hg[bi].wait()
                hw[bi] = pltpu.async_copy(
                    bufs[bi],
                    out_hbm.at[pl.ds((cbase + i) * _CHUNK, _CHUNK)],
                    wsems[bi])
        for i in range(max(0, per_w - _NBUF), per_w):
            hw[i % _NBUF].wait()

    return gather_k


def kernel(doc, table):
    b, l = doc.shape
    n_rows = b * l
    scaled = _renorm_table(table)
    nw = 32
    idx3d = doc.reshape(nw, n_rows // (_CHUNK * nw), _CHUNK)
    out = _make_gather(n_rows, table.shape[0])(scaled, idx3d)
    return out.reshape(b, l, DIM)
